# baseline (speedup 1.0000x reference)
import functools

import jax
import jax.numpy as jnp
from jax import lax
from jax.experimental import pallas as pl
from jax.experimental.pallas import tpu as pltpu
from jax.experimental.pallas import tpu_sc as plsc

_NROW = 16384
_SEQ = 200
_SEQP = 208
_D = 64
_BLK = 16
_RBLK = 1


def _rank_body(ts_ref, tri_ref, idx_ref):
    ts = ts_ref[...]
    b = ts.shape[0]
    tsi = lax.bitcast_convert_type(ts, jnp.int32)
    a_k = jnp.broadcast_to(tsi[:, :, None], (b, _SEQ, _SEQ))
    a_j = jnp.broadcast_to(tsi[:, None, :], (b, _SEQ, _SEQ))
    trif = jnp.broadcast_to(tri_ref[...][None], (b, _SEQ, _SEQ))
    ltf = jnp.where(a_k < a_j, 1.0, 0.0)
    eqf = jnp.where(a_k == a_j, 1.0, 0.0)
    cmp = ltf + eqf * trif
    rank = jnp.sum(cmp, axis=1).astype(jnp.int32)
    pos = rank * _D
    idx_ref[...] = jnp.concatenate(
        [pos, jnp.zeros((b, _SEQP - _SEQ), jnp.int32)], axis=1)


def _tc_rank(ts):
    k_iota = lax.broadcasted_iota(jnp.int32, (_SEQ, _SEQ), 0)
    j_iota = lax.broadcasted_iota(jnp.int32, (_SEQ, _SEQ), 1)
    tri = jnp.where(k_iota < j_iota, 1.0, 0.0).astype(jnp.float32)
    return pl.pallas_call(
        _rank_body,
        grid=(_NROW // _BLK,),
        in_specs=[
            pl.BlockSpec((_BLK, _SEQ), lambda i: (i, 0)),
            pl.BlockSpec((_SEQ, _SEQ), lambda i: (0, 0)),
        ],
        out_specs=pl.BlockSpec((_BLK, _SEQP), lambda i: (i, 0)),
        out_shape=jax.ShapeDtypeStruct((_NROW, _SEQP), jnp.int32),
    )(ts, tri)


def _sc_scatter_body(table_hbm, pos_hbm, out_hbm, tab_v):
    pltpu.sync_copy(table_hbm, tab_v)

    def build(pos_vmem, out_vmem):
        def chunk(jc_static, jc_val, nvalid):
            rv = pos_vmem[0, pl.ds(jc_val * 16, 16)]
            for l in range(nvalid):
                p = rv[l]
                src = (jc_static * 16 + l) * _D if jc_static is not None \
                    else (jc_val * 16 + l) * _D
                for k in range(0, _D, 16):
                    out_vmem[0, pl.ds(p + k, 16)] = tab_v[pl.ds(src + k, 16)]

        @plsc.parallel_loop(0, _SEQ // 16, unroll=2)
        def _(jc):
            chunk(None, jc, 16)

        chunk(_SEQ // 16, _SEQ // 16, _SEQ - (_SEQ // 16) * 16)

    pltpu.emit_pipeline(
        build,
        grid=(_NROW // _RBLK,),
        in_specs=[pl.BlockSpec((_RBLK, _SEQP), lambda i: (i, 0))],
        out_specs=[pl.BlockSpec((_RBLK, _SEQ * _D), lambda i: (i, 0))],
        core_axis_name=("c", "s"),
        dimension_semantics=(pltpu.PARALLEL,),
    )(pos_hbm, out_hbm)


def _sc_scatter(table_flat, pos_arr):
    mesh = plsc.VectorSubcoreMesh(core_axis_name="c", subcore_axis_name="s")
    k = pl.kernel(
        _sc_scatter_body,
        mesh=mesh,
        compiler_params=pltpu.CompilerParams(needs_layout_passes=False),
        out_type=jax.ShapeDtypeStruct((_NROW, _SEQ * _D), jnp.float32),
        scratch_types=[
            pltpu.VMEM((_SEQ * _D,), jnp.float32),
        ],
    )
    return k(table_flat, pos_arr)


@jax.jit
def kernel(ts, pos_embeddings):
    pos = _tc_rank(ts)
    out = _sc_scatter(pos_embeddings.reshape(-1), pos)
    return out.reshape(_NROW, _SEQ, _D)


# speedup vs baseline: 1.0998x; 1.0998x over previous
import functools

import jax
import jax.numpy as jnp
from jax import lax
from jax.experimental import pallas as pl
from jax.experimental.pallas import tpu as pltpu
from jax.experimental.pallas import tpu_sc as plsc

_NROW = 16384
_SEQ = 200
_SEQP = 208
_D = 64
_BLK = 16
_RBLK = 1


def _rank_body(ts_ref, idx_ref):
    ts = ts_ref[...]
    b = ts.shape[0]
    tsi = lax.bitcast_convert_type(ts, jnp.int32)
    a_k = jnp.broadcast_to(tsi[:, :, None], (b, _SEQ, _SEQ))
    a_j = jnp.broadcast_to(tsi[:, None, :], (b, _SEQ, _SEQ))
    k_iota = lax.broadcasted_iota(jnp.int32, (b, _SEQ, _SEQ), 1)
    j_iota = lax.broadcasted_iota(jnp.int32, (b, _SEQ, _SEQ), 2)
    trif = jnp.where(k_iota < j_iota, 1.0, 0.0)
    cmp = jnp.where(a_k < a_j, 1.0, jnp.where(a_k == a_j, trif, 0.0))
    rank = jnp.sum(cmp, axis=1).astype(jnp.int32)
    pos = rank * _D
    idx_ref[...] = jnp.concatenate(
        [pos, jnp.zeros((b, _SEQP - _SEQ), jnp.int32)], axis=1)


def _tc_rank(ts):
    return pl.pallas_call(
        _rank_body,
        grid=(_NROW // _BLK,),
        in_specs=[pl.BlockSpec((_BLK, _SEQ), lambda i: (i, 0))],
        out_specs=pl.BlockSpec((_BLK, _SEQP), lambda i: (i, 0)),
        out_shape=jax.ShapeDtypeStruct((_NROW, _SEQP), jnp.int32),
    )(ts)


def _sc_scatter_body(table_hbm, pos_hbm, out_hbm, tab_v):
    pltpu.sync_copy(table_hbm, tab_v)

    def build(pos_vmem, out_vmem):
        def chunk(jc_static, jc_val, nvalid):
            rv = pos_vmem[0, pl.ds(jc_val * 16, 16)]
            for l in range(nvalid):
                p = rv[l]
                src = (jc_static * 16 + l) * _D if jc_static is not None \
                    else (jc_val * 16 + l) * _D
                for k in range(0, _D, 16):
                    out_vmem[0, pl.ds(p + k, 16)] = tab_v[pl.ds(src + k, 16)]

        @plsc.parallel_loop(0, _SEQ // 16, unroll=2)
        def _(jc):
            chunk(None, jc, 16)

        chunk(_SEQ // 16, _SEQ // 16, _SEQ - (_SEQ // 16) * 16)

    pltpu.emit_pipeline(
        build,
        grid=(_NROW // _RBLK,),
        in_specs=[pl.BlockSpec((_RBLK, _SEQP), lambda i: (i, 0))],
        out_specs=[pl.BlockSpec((_RBLK, _SEQ * _D), lambda i: (i, 0))],
        core_axis_name=("c", "s"),
        dimension_semantics=(pltpu.PARALLEL,),
    )(pos_hbm, out_hbm)


def _sc_scatter(table_flat, pos_arr):
    mesh = plsc.VectorSubcoreMesh(core_axis_name="c", subcore_axis_name="s")
    k = pl.kernel(
        _sc_scatter_body,
        mesh=mesh,
        compiler_params=pltpu.CompilerParams(needs_layout_passes=False),
        out_type=jax.ShapeDtypeStruct((_NROW, _SEQ * _D), jnp.float32),
        scratch_types=[
            pltpu.VMEM((_SEQ * _D,), jnp.float32),
        ],
    )
    return k(table_flat, pos_arr)


@jax.jit
def kernel(ts, pos_embeddings):
    pos = _tc_rank(ts)
    out = _sc_scatter(pos_embeddings.reshape(-1), pos)
    return out.reshape(_NROW, _SEQ, _D)
